# baseline (speedup 1.0000x reference)
import functools

import jax
import jax.numpy as jnp
from jax import lax
from jax.experimental import pallas as pl
from jax.experimental.pallas import tpu as pltpu
from jax.experimental.pallas import tpu_sc as plsc

N = 10000
E = 320000
D = 128
R = 4
B = 1024
H = 64

NC = 2
NS = 16
NW = NC * NS

CH = 128
SH = 40
S = 2 * SH
EPW = S * CH
E_PAD = EPW * NW
N_PAD = 10240
ROWS_PW = N_PAD // NS
NBUF = 2
ZR = 64
K0 = 4


def _mesh():
    return plsc.VectorSubcoreMesh(core_axis_name="c", subcore_axis_name="s")


@functools.partial(
    pl.kernel, mesh=_mesh(),
    out_type=jax.ShapeDtypeStruct((NC * N_PAD, D), jnp.float32),
    scratch_types=(
        [pltpu.VMEM((SH, CH), jnp.int32),
         pltpu.VMEM((SH, CH), jnp.int32),
         pltpu.VMEM((NBUF * CH, D), jnp.float32),
         pltpu.VMEM_SHARED((N_PAD, D), jnp.float32)]
        + [pltpu.SemaphoreType.DMA] * (2 * NBUF)
    ),
)
def _agg(table, gidx3, dst3, acc_out, gidx_v, dst_v, rows_v, acc_sh, *sems):
    semg = sems[:NBUF]
    sems_ = sems[NBUF:]
    cid = lax.axis_index("c")
    sid = lax.axis_index("s")

    def zrow(j, c):
        for k in range(D // 16):
            rows_v[j, pl.ds(k * 16, 16)] = jnp.zeros((16,), jnp.float32)
        return c
    lax.fori_loop(0, ZR, zrow, 0)
    base = sid * ROWS_PW
    for t in range(ROWS_PW // ZR):
        pltpu.sync_copy(rows_v.at[pl.ds(0, ZR)],
                        acc_sh.at[pl.ds(base + t * ZR, ZR)])
    plsc.subcore_barrier()

    def buf(b):
        return rows_v.at[pl.ds(b * CH, CH)]

    def fire_g(b, j):
        pltpu.async_copy(table.at[gidx_v.at[j]], buf(b), semg[b])

    def wait_g(b, j):
        pltpu.make_async_copy(table.at[gidx_v.at[j]], buf(b), semg[b]).wait()

    def fire_s(b, j):
        pltpu.async_copy(buf(b), acc_sh.at[dst_v.at[j]], sems_[b], add=True)

    def wait_s(b, j):
        pltpu.make_async_copy(buf(b), acc_sh.at[dst_v.at[j]],
                              sems_[b]).wait()

    def run_slice(hidx):
        pltpu.sync_copy(gidx3.at[hidx], gidx_v)
        pltpu.sync_copy(dst3.at[hidx], dst_v)
        for b in range(NBUF):
            fire_g(b, b)

        def group(t, c):
            for b in range(NBUF):
                j = t * NBUF + b
                wait_g(b, j)
                fire_s(b, j)
            for b in range(NBUF):
                jn = (t + 1) * NBUF + b
                wait_s(b, jn)
                fire_g(b, jn)
            return c
        lax.fori_loop(0, SH // NBUF - 1, group, 0)

        for b in range(NBUF):
            j = SH - NBUF + b
            wait_g(b, j)
            fire_s(b, j)
        for b in range(NBUF):
            wait_s(b, 0)

    @pl.when(cid == 0)
    def _core0():
        for q in range(K0):
            run_slice(sid * K0 + q)

    if K0 < 4:
        @pl.when(cid == 1)
        def _core1():
            for q in range(4 - K0):
                run_slice(NS * K0 + sid * (4 - K0) + q)

    plsc.subcore_barrier()

    pltpu.sync_copy(acc_sh.at[pl.ds(base, ROWS_PW)],
                    acc_out.at[pl.ds(cid * N_PAD + base, ROWS_PW)])


@functools.partial(
    pl.kernel, mesh=_mesh(),
    out_type=jax.ShapeDtypeStruct((NC * N_PAD,), jnp.float32),
    scratch_types=[
        pltpu.VMEM((S, CH), jnp.int32),
        pltpu.VMEM((CH,), jnp.float32),
        pltpu.VMEM((ZR,), jnp.float32),
        pltpu.VMEM_SHARED((N_PAD,), jnp.float32),
        pltpu.SemaphoreType.DMA,
    ],
)
def _deg(dst3, deg_out, dst_v, ones_v, zeros_v, deg_sh, semd):
    cid = lax.axis_index("c")
    sid = lax.axis_index("s")
    wid = sid * NC + cid
    for h in range(2):
        pltpu.sync_copy(dst3.at[2 * wid + h], dst_v.at[pl.ds(h * SH, SH)])
    for k in range(CH // 16):
        ones_v[pl.ds(k * 16, 16)] = jnp.ones((16,), jnp.float32)
    for k in range(ZR // 16):
        zeros_v[pl.ds(k * 16, 16)] = jnp.zeros((16,), jnp.float32)
    base = sid * ROWS_PW
    for t in range(ROWS_PW // ZR):
        pltpu.sync_copy(zeros_v, deg_sh.at[pl.ds(base + t * ZR, ZR)])
    plsc.subcore_barrier()

    def step(j, c):
        pltpu.async_copy(ones_v, deg_sh.at[dst_v.at[j]], semd, add=True)
        return c
    lax.fori_loop(0, S, step, 0)

    def drain(j, c):
        pltpu.make_async_copy(ones_v, deg_sh.at[dst_v.at[0]], semd).wait()
        return c
    lax.fori_loop(0, S, drain, 0)

    plsc.subcore_barrier()
    pltpu.sync_copy(deg_sh.at[pl.ds(base, ROWS_PW)],
                    deg_out.at[pl.ds(cid * N_PAD + base, ROWS_PW)])


_BPW = (2 * B) // NW


@functools.partial(
    pl.kernel, mesh=_mesh(),
    out_type=jax.ShapeDtypeStruct((2 * B, D), jnp.float32),
    scratch_types=[
        pltpu.VMEM((_BPW,), jnp.int32),
        pltpu.VMEM((_BPW, D), jnp.float32),
        pltpu.SemaphoreType.DMA,
    ],
)
def _pair_gather(h2, idx, out, idx_v, rows_v, sem):
    wid = lax.axis_index("s") * NC + lax.axis_index("c")
    base = wid * _BPW
    pltpu.sync_copy(idx.at[pl.ds(base, _BPW)], idx_v)
    pltpu.async_copy(h2.at[idx_v], rows_v, sem).wait()
    pltpu.sync_copy(rows_v, out.at[pl.ds(base, _BPW)])


def _transform_body(x_ref, wr_ref, ws_ref, hrel_ref, hs_ref):
    xb = x_ref[...]
    for r in range(R):
        hrel_ref[r] = jnp.dot(xb, wr_ref[r], preferred_element_type=jnp.float32)
    hs_ref[...] = jnp.dot(xb, ws_ref[...], preferred_element_type=jnp.float32)


def _tc_transform(x, W_rel, W_self):
    bn = 2000
    return pl.pallas_call(
        _transform_body,
        grid=(N // bn,),
        in_specs=[
            pl.BlockSpec((bn, D), lambda i: (i, 0)),
            pl.BlockSpec((R, D, D), lambda i: (0, 0, 0)),
            pl.BlockSpec((D, D), lambda i: (0, 0)),
        ],
        out_specs=[
            pl.BlockSpec((R, bn, D), lambda i: (0, i, 0)),
            pl.BlockSpec((bn, D), lambda i: (i, 0)),
        ],
        out_shape=[
            jax.ShapeDtypeStruct((R, N, D), jnp.float32),
            jax.ShapeDtypeStruct((N, D), jnp.float32),
        ],
    )(x, W_rel, W_self)


def _combine1_body(acc_ref, deg_ref, hs1_ref, wr2_ref, ws2_ref,
                   hrel2_ref, hs2_ref):
    agg = acc_ref[0] + acc_ref[1]
    deg = jnp.maximum(deg_ref[0] + deg_ref[1], 1.0)
    h = jnp.maximum(agg / deg[:, None] + hs1_ref[...], 0.0)
    for r in range(R):
        hrel2_ref[r] = jnp.dot(h, wr2_ref[r], preferred_element_type=jnp.float32)
    hs2_ref[...] = jnp.dot(h, ws2_ref[...], preferred_element_type=jnp.float32)


def _tc_combine1(acc, deg, hs1, W_rel2, W_self2):
    bn = 2048
    g = N_PAD // bn
    return pl.pallas_call(
        _combine1_body,
        grid=(g,),
        in_specs=[
            pl.BlockSpec((NC, bn, D), lambda i: (0, i, 0)),
            pl.BlockSpec((NC, bn), lambda i: (0, i)),
            pl.BlockSpec((bn, D), lambda i: (i, 0)),
            pl.BlockSpec((R, D, D), lambda i: (0, 0, 0)),
            pl.BlockSpec((D, D), lambda i: (0, 0)),
        ],
        out_specs=[
            pl.BlockSpec((R, bn, D), lambda i: (0, i, 0)),
            pl.BlockSpec((bn, D), lambda i: (i, 0)),
        ],
        out_shape=[
            jax.ShapeDtypeStruct((R, N, D), jnp.float32),
            jax.ShapeDtypeStruct((N, D), jnp.float32),
        ],
    )(acc, deg, hs1, W_rel2, W_self2)


def _combine2_body(acc_ref, deg_ref, hs2_ref, h2_ref):
    agg = acc_ref[0] + acc_ref[1]
    deg = jnp.maximum(deg_ref[0] + deg_ref[1], 1.0)
    h2_ref[...] = agg / deg[:, None] + hs2_ref[...]


def _tc_combine2(acc, deg, hs2):
    bn = 2048
    g = N_PAD // bn
    return pl.pallas_call(
        _combine2_body,
        grid=(g,),
        in_specs=[
            pl.BlockSpec((NC, bn, D), lambda i: (0, i, 0)),
            pl.BlockSpec((NC, bn), lambda i: (0, i)),
            pl.BlockSpec((bn, D), lambda i: (i, 0)),
        ],
        out_specs=pl.BlockSpec((bn, D), lambda i: (i, 0)),
        out_shape=jax.ShapeDtypeStruct((N, D), jnp.float32),
    )(acc, deg, hs2)


def _head_body(e_ref, wa_ref, va_ref, wm1_ref, bm1_ref, wm2_ref, bm2_ref,
               out_ref):
    e = e_ref[...]
    s = jnp.tanh(jnp.dot(e, wa_ref[...], preferred_element_type=jnp.float32))
    score = jnp.dot(s, va_ref[...], preferred_element_type=jnp.float32)
    gate = jax.nn.sigmoid(score)
    w = gate * e
    o = jnp.dot(w, wm1_ref[...], preferred_element_type=jnp.float32)
    o = jnp.maximum(o + bm1_ref[...], 0.0)
    o = jnp.dot(o, wm2_ref[...], preferred_element_type=jnp.float32)
    o = o + bm2_ref[...]
    out_ref[...] = jnp.mean(o, axis=0, keepdims=True)


def _tc_head(e, Wa, va, Wm1, bm1, Wm2, bm2):
    return pl.pallas_call(
        _head_body,
        out_shape=jax.ShapeDtypeStruct((1, 1), jnp.float32),
    )(e, Wa, va, Wm1, bm1, Wm2, bm2)


def kernel(x, edge_index, edge_type, user_item_pairs,
           W_rel1, W_self1, W_rel2, W_self2, Wa, va, Wm1, bm1, Wm2, bm2):
    src = edge_index[0].astype(jnp.int32)
    dst = edge_index[1].astype(jnp.int32)
    et = edge_type.astype(jnp.int32)
    gidx = et * N + src
    pad = E_PAD - E
    zpad = jnp.zeros((pad,), jnp.int32)
    jpad = N + jnp.arange(pad, dtype=jnp.int32) % (N_PAD - N)
    gidx3 = jnp.concatenate([gidx, zpad]).reshape(2 * NW, SH, CH)
    dst3 = jnp.concatenate([dst, jpad]).reshape(2 * NW, SH, CH)

    hrel1, hs1 = _tc_transform(x, W_rel1, W_self1)
    deg = _deg(dst3).reshape(NC, N_PAD)
    acc1 = _agg(hrel1.reshape(R * N, D), gidx3, dst3)
    acc1 = acc1.reshape(NC, N_PAD, D)
    hrel2, hs2 = _tc_combine1(acc1, deg, hs1, W_rel2, W_self2)
    acc2 = _agg(hrel2.reshape(R * N, D), gidx3, dst3)
    acc2 = acc2.reshape(NC, N_PAD, D)
    h2 = _tc_combine2(acc2, deg, hs2)

    pidx = user_item_pairs.astype(jnp.int32).reshape(2 * B)
    rows = _pair_gather(h2, pidx)
    e = rows.reshape(B, 2 * D)
    out = _tc_head(e, Wa, va.reshape(H, 1), Wm1, bm1.reshape(1, H),
                   Wm2, bm2.reshape(1, 1))
    return out.reshape(1)


# speedup vs baseline: 3.5573x; 3.5573x over previous
import functools

import jax
import jax.numpy as jnp
from jax import lax
from jax.experimental import pallas as pl
from jax.experimental.pallas import tpu as pltpu
from jax.experimental.pallas import tpu_sc as plsc

N = 10000
E = 320000
D = 128
R = 4
B = 1024
H = 64

NC = 2
NS = 16
NW = NC * NS

CH = 128
SH = 40
S = 2 * SH
EPW = S * CH
E_PAD = EPW * NW
N_PAD = 10240
ROWS_PW = N_PAD // NS
NBUF = 2
ZR = 64
K0 = 2


def _mesh():
    return plsc.VectorSubcoreMesh(core_axis_name="c", subcore_axis_name="s")


@functools.partial(
    pl.kernel, mesh=_mesh(),
    out_type=jax.ShapeDtypeStruct((NC * N_PAD, D), jnp.float32),
    scratch_types=(
        [pltpu.VMEM((SH, CH), jnp.int32),
         pltpu.VMEM((SH, CH), jnp.int32),
         pltpu.VMEM((NBUF * CH, D), jnp.float32),
         pltpu.VMEM_SHARED((N_PAD, D), jnp.float32)]
        + [pltpu.SemaphoreType.DMA] * (2 * NBUF)
    ),
)
def _agg(table, gidx3, dst3, acc_out, gidx_v, dst_v, rows_v, acc_sh, *sems):
    semg = sems[:NBUF]
    sems_ = sems[NBUF:]
    cid = lax.axis_index("c")
    sid = lax.axis_index("s")

    def zrow(j, c):
        for k in range(D // 16):
            rows_v[j, pl.ds(k * 16, 16)] = jnp.zeros((16,), jnp.float32)
        return c
    lax.fori_loop(0, ZR, zrow, 0)
    base = sid * ROWS_PW
    for t in range(ROWS_PW // ZR):
        pltpu.sync_copy(rows_v.at[pl.ds(0, ZR)],
                        acc_sh.at[pl.ds(base + t * ZR, ZR)])
    plsc.subcore_barrier()

    def buf(b):
        return rows_v.at[pl.ds(b * CH, CH)]

    def fire_g(b, j):
        pltpu.async_copy(table.at[gidx_v.at[j]], buf(b), semg[b])

    def wait_g(b, j):
        pltpu.make_async_copy(table.at[gidx_v.at[j]], buf(b), semg[b]).wait()

    def fire_s(b, j):
        pltpu.async_copy(buf(b), acc_sh.at[dst_v.at[j]], sems_[b], add=True)

    def wait_s(b, j):
        pltpu.make_async_copy(buf(b), acc_sh.at[dst_v.at[j]],
                              sems_[b]).wait()

    def run_slice(hidx):
        pltpu.sync_copy(gidx3.at[hidx], gidx_v)
        pltpu.sync_copy(dst3.at[hidx], dst_v)
        for b in range(NBUF):
            fire_g(b, b)

        def group(t, c):
            for b in range(NBUF):
                j = t * NBUF + b
                wait_g(b, j)
                fire_s(b, j)
            for b in range(NBUF):
                jn = (t + 1) * NBUF + b
                wait_s(b, jn)
                fire_g(b, jn)
            return c
        lax.fori_loop(0, SH // NBUF - 1, group, 0)

        for b in range(NBUF):
            j = SH - NBUF + b
            wait_g(b, j)
            fire_s(b, j)
        for b in range(NBUF):
            wait_s(b, 0)

    @pl.when(cid == 0)
    def _core0():
        for q in range(K0):
            run_slice(sid * K0 + q)

    if K0 < 4:
        @pl.when(cid == 1)
        def _core1():
            for q in range(4 - K0):
                run_slice(NS * K0 + sid * (4 - K0) + q)

    plsc.subcore_barrier()

    pltpu.sync_copy(acc_sh.at[pl.ds(base, ROWS_PW)],
                    acc_out.at[pl.ds(cid * N_PAD + base, ROWS_PW)])


@functools.partial(
    pl.kernel, mesh=_mesh(),
    out_type=jax.ShapeDtypeStruct((NC * N_PAD,), jnp.float32),
    scratch_types=[
        pltpu.VMEM((S, CH), jnp.int32),
        pltpu.VMEM((CH,), jnp.float32),
        pltpu.VMEM((ZR,), jnp.float32),
        pltpu.VMEM_SHARED((N_PAD,), jnp.float32),
        pltpu.SemaphoreType.DMA,
    ],
)
def _deg(dst3, deg_out, dst_v, ones_v, zeros_v, deg_sh, semd):
    cid = lax.axis_index("c")
    sid = lax.axis_index("s")
    wid = sid * NC + cid
    for h in range(2):
        pltpu.sync_copy(dst3.at[2 * wid + h], dst_v.at[pl.ds(h * SH, SH)])
    for k in range(CH // 16):
        ones_v[pl.ds(k * 16, 16)] = jnp.ones((16,), jnp.float32)
    for k in range(ZR // 16):
        zeros_v[pl.ds(k * 16, 16)] = jnp.zeros((16,), jnp.float32)
    base = sid * ROWS_PW
    for t in range(ROWS_PW // ZR):
        pltpu.sync_copy(zeros_v, deg_sh.at[pl.ds(base + t * ZR, ZR)])
    plsc.subcore_barrier()

    def step(j, c):
        pltpu.async_copy(ones_v, deg_sh.at[dst_v.at[j]], semd, add=True)
        return c
    lax.fori_loop(0, S, step, 0)

    def drain(j, c):
        pltpu.make_async_copy(ones_v, deg_sh.at[dst_v.at[0]], semd).wait()
        return c
    lax.fori_loop(0, S, drain, 0)

    plsc.subcore_barrier()
    pltpu.sync_copy(deg_sh.at[pl.ds(base, ROWS_PW)],
                    deg_out.at[pl.ds(cid * N_PAD + base, ROWS_PW)])


_BPW = (2 * B) // NW


@functools.partial(
    pl.kernel, mesh=_mesh(),
    out_type=jax.ShapeDtypeStruct((2 * B, D), jnp.float32),
    scratch_types=[
        pltpu.VMEM((_BPW,), jnp.int32),
        pltpu.VMEM((_BPW, D), jnp.float32),
        pltpu.SemaphoreType.DMA,
    ],
)
def _pair_gather(h2, idx, out, idx_v, rows_v, sem):
    wid = lax.axis_index("s") * NC + lax.axis_index("c")
    base = wid * _BPW
    pltpu.sync_copy(idx.at[pl.ds(base, _BPW)], idx_v)
    pltpu.async_copy(h2.at[idx_v], rows_v, sem).wait()
    pltpu.sync_copy(rows_v, out.at[pl.ds(base, _BPW)])


def _transform_body(x_ref, wr_ref, ws_ref, hrel_ref, hs_ref):
    xb = x_ref[...]
    for r in range(R):
        hrel_ref[r] = jnp.dot(xb, wr_ref[r], preferred_element_type=jnp.float32)
    hs_ref[...] = jnp.dot(xb, ws_ref[...], preferred_element_type=jnp.float32)


def _tc_transform(x, W_rel, W_self):
    bn = 2000
    return pl.pallas_call(
        _transform_body,
        grid=(N // bn,),
        in_specs=[
            pl.BlockSpec((bn, D), lambda i: (i, 0)),
            pl.BlockSpec((R, D, D), lambda i: (0, 0, 0)),
            pl.BlockSpec((D, D), lambda i: (0, 0)),
        ],
        out_specs=[
            pl.BlockSpec((R, bn, D), lambda i: (0, i, 0)),
            pl.BlockSpec((bn, D), lambda i: (i, 0)),
        ],
        out_shape=[
            jax.ShapeDtypeStruct((R, N, D), jnp.float32),
            jax.ShapeDtypeStruct((N, D), jnp.float32),
        ],
    )(x, W_rel, W_self)


def _combine1_body(acc_ref, deg_ref, hs1_ref, wr2_ref, ws2_ref,
                   hrel2_ref, hs2_ref):
    agg = acc_ref[0] + acc_ref[1]
    deg = jnp.maximum(deg_ref[0] + deg_ref[1], 1.0)
    h = jnp.maximum(agg / deg[:, None] + hs1_ref[...], 0.0)
    for r in range(R):
        hrel2_ref[r] = jnp.dot(h, wr2_ref[r], preferred_element_type=jnp.float32)
    hs2_ref[...] = jnp.dot(h, ws2_ref[...], preferred_element_type=jnp.float32)


def _tc_combine1(acc, deg, hs1, W_rel2, W_self2):
    bn = 2048
    g = N_PAD // bn
    return pl.pallas_call(
        _combine1_body,
        grid=(g,),
        in_specs=[
            pl.BlockSpec((NC, bn, D), lambda i: (0, i, 0)),
            pl.BlockSpec((NC, bn), lambda i: (0, i)),
            pl.BlockSpec((bn, D), lambda i: (i, 0)),
            pl.BlockSpec((R, D, D), lambda i: (0, 0, 0)),
            pl.BlockSpec((D, D), lambda i: (0, 0)),
        ],
        out_specs=[
            pl.BlockSpec((R, bn, D), lambda i: (0, i, 0)),
            pl.BlockSpec((bn, D), lambda i: (i, 0)),
        ],
        out_shape=[
            jax.ShapeDtypeStruct((R, N, D), jnp.float32),
            jax.ShapeDtypeStruct((N, D), jnp.float32),
        ],
    )(acc, deg, hs1, W_rel2, W_self2)


def _combine2_body(acc_ref, deg_ref, hs2_ref, h2_ref):
    agg = acc_ref[0] + acc_ref[1]
    deg = jnp.maximum(deg_ref[0] + deg_ref[1], 1.0)
    h2_ref[...] = agg / deg[:, None] + hs2_ref[...]


def _tc_combine2(acc, deg, hs2):
    bn = 2048
    g = N_PAD // bn
    return pl.pallas_call(
        _combine2_body,
        grid=(g,),
        in_specs=[
            pl.BlockSpec((NC, bn, D), lambda i: (0, i, 0)),
            pl.BlockSpec((NC, bn), lambda i: (0, i)),
            pl.BlockSpec((bn, D), lambda i: (i, 0)),
        ],
        out_specs=pl.BlockSpec((bn, D), lambda i: (i, 0)),
        out_shape=jax.ShapeDtypeStruct((N, D), jnp.float32),
    )(acc, deg, hs2)


def _head_body(e_ref, wa_ref, va_ref, wm1_ref, bm1_ref, wm2_ref, bm2_ref,
               out_ref):
    e = e_ref[...]
    s = jnp.tanh(jnp.dot(e, wa_ref[...], preferred_element_type=jnp.float32))
    score = jnp.dot(s, va_ref[...], preferred_element_type=jnp.float32)
    gate = jax.nn.sigmoid(score)
    w = gate * e
    o = jnp.dot(w, wm1_ref[...], preferred_element_type=jnp.float32)
    o = jnp.maximum(o + bm1_ref[...], 0.0)
    o = jnp.dot(o, wm2_ref[...], preferred_element_type=jnp.float32)
    o = o + bm2_ref[...]
    out_ref[...] = jnp.mean(o, axis=0, keepdims=True)


def _tc_head(e, Wa, va, Wm1, bm1, Wm2, bm2):
    return pl.pallas_call(
        _head_body,
        out_shape=jax.ShapeDtypeStruct((1, 1), jnp.float32),
    )(e, Wa, va, Wm1, bm1, Wm2, bm2)


def kernel(x, edge_index, edge_type, user_item_pairs,
           W_rel1, W_self1, W_rel2, W_self2, Wa, va, Wm1, bm1, Wm2, bm2):
    src = edge_index[0].astype(jnp.int32)
    dst = edge_index[1].astype(jnp.int32)
    et = edge_type.astype(jnp.int32)
    gidx = et * N + src
    pad = E_PAD - E
    gpad = jnp.arange(pad, dtype=jnp.int32) % N
    jpad = N + jnp.arange(pad, dtype=jnp.int32) % (N_PAD - N)
    gidx3 = jnp.concatenate([gidx, gpad]).reshape(2 * NW, SH, CH)
    dst3 = jnp.concatenate([dst, jpad]).reshape(2 * NW, SH, CH)

    hrel1, hs1 = _tc_transform(x, W_rel1, W_self1)
    deg = _deg(dst3).reshape(NC, N_PAD)
    acc1 = _agg(hrel1.reshape(R * N, D), gidx3, dst3)
    acc1 = acc1.reshape(NC, N_PAD, D)
    hrel2, hs2 = _tc_combine1(acc1, deg, hs1, W_rel2, W_self2)
    acc2 = _agg(hrel2.reshape(R * N, D), gidx3, dst3)
    acc2 = acc2.reshape(NC, N_PAD, D)
    h2 = _tc_combine2(acc2, deg, hs2)

    pidx = user_item_pairs.astype(jnp.int32).reshape(2 * B)
    rows = _pair_gather(h2, pidx)
    e = rows.reshape(B, 2 * D)
    out = _tc_head(e, Wa, va.reshape(H, 1), Wm1, bm1.reshape(1, H),
                   Wm2, bm2.reshape(1, 1))
    return out.reshape(1)


# speedup vs baseline: 4.0182x; 1.1296x over previous
import functools

import jax
import jax.numpy as jnp
from jax import lax
from jax.experimental import pallas as pl
from jax.experimental.pallas import tpu as pltpu
from jax.experimental.pallas import tpu_sc as plsc

N = 10000
E = 320000
D = 128
R = 4
B = 1024
H = 64

NC = 2
NS = 16
NW = NC * NS

CH = 80
SH = 63
S = 2 * SH
EPW = S * CH
E_PAD = EPW * NW
N_PAD = 10240
ROWS_PW = N_PAD // NS
NBUF = 3
ZR = 64
K0 = 2


def _mesh():
    return plsc.VectorSubcoreMesh(core_axis_name="c", subcore_axis_name="s")


@functools.partial(
    pl.kernel, mesh=_mesh(),
    out_type=jax.ShapeDtypeStruct((NC * N_PAD, D), jnp.float32),
    scratch_types=(
        [pltpu.VMEM((SH, CH), jnp.int32),
         pltpu.VMEM((SH, CH), jnp.int32),
         pltpu.VMEM((NBUF * CH, D), jnp.float32),
         pltpu.VMEM_SHARED((N_PAD, D), jnp.float32)]
        + [pltpu.SemaphoreType.DMA] * (2 * NBUF)
    ),
)
def _agg(table, gidx3, dst3, acc_out, gidx_v, dst_v, rows_v, acc_sh, *sems):
    semg = sems[:NBUF]
    sems_ = sems[NBUF:]
    cid = lax.axis_index("c")
    sid = lax.axis_index("s")

    def zrow(j, c):
        for k in range(D // 16):
            rows_v[j, pl.ds(k * 16, 16)] = jnp.zeros((16,), jnp.float32)
        return c
    lax.fori_loop(0, ZR, zrow, 0)
    base = sid * ROWS_PW
    for t in range(ROWS_PW // ZR):
        pltpu.sync_copy(rows_v.at[pl.ds(0, ZR)],
                        acc_sh.at[pl.ds(base + t * ZR, ZR)])
    plsc.subcore_barrier()

    def buf(b):
        return rows_v.at[pl.ds(b * CH, CH)]

    def fire_g(b, j):
        pltpu.async_copy(table.at[gidx_v.at[j]], buf(b), semg[b])

    def wait_g(b, j):
        pltpu.make_async_copy(table.at[gidx_v.at[j]], buf(b), semg[b]).wait()

    def fire_s(b, j):
        pltpu.async_copy(buf(b), acc_sh.at[dst_v.at[j]], sems_[b], add=True)

    def wait_s(b, j):
        pltpu.make_async_copy(buf(b), acc_sh.at[dst_v.at[j]],
                              sems_[b]).wait()

    def run_slice(hidx):
        pltpu.sync_copy(gidx3.at[hidx], gidx_v)
        pltpu.sync_copy(dst3.at[hidx], dst_v)
        for b in range(NBUF):
            fire_g(b, b)

        def group(t, c):
            for b in range(NBUF):
                j = t * NBUF + b
                wait_g(b, j)
                fire_s(b, j)
            for b in range(NBUF):
                jn = (t + 1) * NBUF + b
                wait_s(b, jn)
                fire_g(b, jn)
            return c
        lax.fori_loop(0, SH // NBUF - 1, group, 0)

        for b in range(NBUF):
            j = SH - NBUF + b
            wait_g(b, j)
            fire_s(b, j)
        for b in range(NBUF):
            wait_s(b, 0)

    @pl.when(cid == 0)
    def _core0():
        for q in range(K0):
            run_slice(sid * K0 + q)

    if K0 < 4:
        @pl.when(cid == 1)
        def _core1():
            for q in range(4 - K0):
                run_slice(NS * K0 + sid * (4 - K0) + q)

    plsc.subcore_barrier()

    pltpu.sync_copy(acc_sh.at[pl.ds(base, ROWS_PW)],
                    acc_out.at[pl.ds(cid * N_PAD + base, ROWS_PW)])


@functools.partial(
    pl.kernel, mesh=_mesh(),
    out_type=jax.ShapeDtypeStruct((NC * N_PAD,), jnp.float32),
    scratch_types=[
        pltpu.VMEM((S, CH), jnp.int32),
        pltpu.VMEM((CH,), jnp.float32),
        pltpu.VMEM((ZR,), jnp.float32),
        pltpu.VMEM_SHARED((N_PAD,), jnp.float32),
        pltpu.SemaphoreType.DMA,
    ],
)
def _deg(dst3, deg_out, dst_v, ones_v, zeros_v, deg_sh, semd):
    cid = lax.axis_index("c")
    sid = lax.axis_index("s")
    wid = sid * NC + cid
    for h in range(2):
        pltpu.sync_copy(dst3.at[2 * wid + h], dst_v.at[pl.ds(h * SH, SH)])
    for k in range(CH // 16):
        ones_v[pl.ds(k * 16, 16)] = jnp.ones((16,), jnp.float32)
    for k in range(ZR // 16):
        zeros_v[pl.ds(k * 16, 16)] = jnp.zeros((16,), jnp.float32)
    base = sid * ROWS_PW
    for t in range(ROWS_PW // ZR):
        pltpu.sync_copy(zeros_v, deg_sh.at[pl.ds(base + t * ZR, ZR)])
    plsc.subcore_barrier()

    def step(j, c):
        pltpu.async_copy(ones_v, deg_sh.at[dst_v.at[j]], semd, add=True)
        return c
    lax.fori_loop(0, S, step, 0)

    def drain(j, c):
        pltpu.make_async_copy(ones_v, deg_sh.at[dst_v.at[0]], semd).wait()
        return c
    lax.fori_loop(0, S, drain, 0)

    plsc.subcore_barrier()
    pltpu.sync_copy(deg_sh.at[pl.ds(base, ROWS_PW)],
                    deg_out.at[pl.ds(cid * N_PAD + base, ROWS_PW)])


_BPW = (2 * B) // NW


@functools.partial(
    pl.kernel, mesh=_mesh(),
    out_type=jax.ShapeDtypeStruct((2 * B, D), jnp.float32),
    scratch_types=[
        pltpu.VMEM((_BPW,), jnp.int32),
        pltpu.VMEM((_BPW, D), jnp.float32),
        pltpu.SemaphoreType.DMA,
    ],
)
def _pair_gather(h2, idx, out, idx_v, rows_v, sem):
    wid = lax.axis_index("s") * NC + lax.axis_index("c")
    base = wid * _BPW
    pltpu.sync_copy(idx.at[pl.ds(base, _BPW)], idx_v)
    pltpu.async_copy(h2.at[idx_v], rows_v, sem).wait()
    pltpu.sync_copy(rows_v, out.at[pl.ds(base, _BPW)])


def _transform_body(x_ref, wr_ref, ws_ref, hrel_ref, hs_ref):
    xb = x_ref[...]
    for r in range(R):
        hrel_ref[r] = jnp.dot(xb, wr_ref[r], preferred_element_type=jnp.float32)
    hs_ref[...] = jnp.dot(xb, ws_ref[...], preferred_element_type=jnp.float32)


def _tc_transform(x, W_rel, W_self):
    bn = 2000
    return pl.pallas_call(
        _transform_body,
        grid=(N // bn,),
        in_specs=[
            pl.BlockSpec((bn, D), lambda i: (i, 0)),
            pl.BlockSpec((R, D, D), lambda i: (0, 0, 0)),
            pl.BlockSpec((D, D), lambda i: (0, 0)),
        ],
        out_specs=[
            pl.BlockSpec((R, bn, D), lambda i: (0, i, 0)),
            pl.BlockSpec((bn, D), lambda i: (i, 0)),
        ],
        out_shape=[
            jax.ShapeDtypeStruct((R, N, D), jnp.float32),
            jax.ShapeDtypeStruct((N, D), jnp.float32),
        ],
    )(x, W_rel, W_self)


def _combine1_body(acc_ref, deg_ref, hs1_ref, wr2_ref, ws2_ref,
                   hrel2_ref, hs2_ref):
    agg = acc_ref[0] + acc_ref[1]
    deg = jnp.maximum(deg_ref[0] + deg_ref[1], 1.0)
    h = jnp.maximum(agg / deg[:, None] + hs1_ref[...], 0.0)
    for r in range(R):
        hrel2_ref[r] = jnp.dot(h, wr2_ref[r], preferred_element_type=jnp.float32)
    hs2_ref[...] = jnp.dot(h, ws2_ref[...], preferred_element_type=jnp.float32)


def _tc_combine1(acc, deg, hs1, W_rel2, W_self2):
    bn = 2048
    g = N_PAD // bn
    return pl.pallas_call(
        _combine1_body,
        grid=(g,),
        in_specs=[
            pl.BlockSpec((NC, bn, D), lambda i: (0, i, 0)),
            pl.BlockSpec((NC, bn), lambda i: (0, i)),
            pl.BlockSpec((bn, D), lambda i: (i, 0)),
            pl.BlockSpec((R, D, D), lambda i: (0, 0, 0)),
            pl.BlockSpec((D, D), lambda i: (0, 0)),
        ],
        out_specs=[
            pl.BlockSpec((R, bn, D), lambda i: (0, i, 0)),
            pl.BlockSpec((bn, D), lambda i: (i, 0)),
        ],
        out_shape=[
            jax.ShapeDtypeStruct((R, N, D), jnp.float32),
            jax.ShapeDtypeStruct((N, D), jnp.float32),
        ],
    )(acc, deg, hs1, W_rel2, W_self2)


def _combine2_body(acc_ref, deg_ref, hs2_ref, h2_ref):
    agg = acc_ref[0] + acc_ref[1]
    deg = jnp.maximum(deg_ref[0] + deg_ref[1], 1.0)
    h2_ref[...] = agg / deg[:, None] + hs2_ref[...]


def _tc_combine2(acc, deg, hs2):
    bn = 2048
    g = N_PAD // bn
    return pl.pallas_call(
        _combine2_body,
        grid=(g,),
        in_specs=[
            pl.BlockSpec((NC, bn, D), lambda i: (0, i, 0)),
            pl.BlockSpec((NC, bn), lambda i: (0, i)),
            pl.BlockSpec((bn, D), lambda i: (i, 0)),
        ],
        out_specs=pl.BlockSpec((bn, D), lambda i: (i, 0)),
        out_shape=jax.ShapeDtypeStruct((N, D), jnp.float32),
    )(acc, deg, hs2)


def _head_body(e_ref, wa_ref, va_ref, wm1_ref, bm1_ref, wm2_ref, bm2_ref,
               out_ref):
    e = e_ref[...]
    s = jnp.tanh(jnp.dot(e, wa_ref[...], preferred_element_type=jnp.float32))
    score = jnp.dot(s, va_ref[...], preferred_element_type=jnp.float32)
    gate = jax.nn.sigmoid(score)
    w = gate * e
    o = jnp.dot(w, wm1_ref[...], preferred_element_type=jnp.float32)
    o = jnp.maximum(o + bm1_ref[...], 0.0)
    o = jnp.dot(o, wm2_ref[...], preferred_element_type=jnp.float32)
    o = o + bm2_ref[...]
    out_ref[...] = jnp.mean(o, axis=0, keepdims=True)


def _tc_head(e, Wa, va, Wm1, bm1, Wm2, bm2):
    return pl.pallas_call(
        _head_body,
        out_shape=jax.ShapeDtypeStruct((1, 1), jnp.float32),
    )(e, Wa, va, Wm1, bm1, Wm2, bm2)


def kernel(x, edge_index, edge_type, user_item_pairs,
           W_rel1, W_self1, W_rel2, W_self2, Wa, va, Wm1, bm1, Wm2, bm2):
    src = edge_index[0].astype(jnp.int32)
    dst = edge_index[1].astype(jnp.int32)
    et = edge_type.astype(jnp.int32)
    gidx = et * N + src
    pad = E_PAD - E
    gpad = jnp.arange(pad, dtype=jnp.int32) % N
    jpad = N + jnp.arange(pad, dtype=jnp.int32) % (N_PAD - N)
    gidx3 = jnp.concatenate([gidx, gpad]).reshape(2 * NW, SH, CH)
    dst3 = jnp.concatenate([dst, jpad]).reshape(2 * NW, SH, CH)

    hrel1, hs1 = _tc_transform(x, W_rel1, W_self1)
    deg = _deg(dst3).reshape(NC, N_PAD)
    acc1 = _agg(hrel1.reshape(R * N, D), gidx3, dst3)
    acc1 = acc1.reshape(NC, N_PAD, D)
    hrel2, hs2 = _tc_combine1(acc1, deg, hs1, W_rel2, W_self2)
    acc2 = _agg(hrel2.reshape(R * N, D), gidx3, dst3)
    acc2 = acc2.reshape(NC, N_PAD, D)
    h2 = _tc_combine2(acc2, deg, hs2)

    pidx = user_item_pairs.astype(jnp.int32).reshape(2 * B)
    rows = _pair_gather(h2, pidx)
    e = rows.reshape(B, 2 * D)
    out = _tc_head(e, Wa, va.reshape(H, 1), Wm1, bm1.reshape(1, H),
                   Wm2, bm2.reshape(1, 1))
    return out.reshape(1)


# speedup vs baseline: 4.0649x; 1.0116x over previous
import functools

import jax
import jax.numpy as jnp
from jax import lax
from jax.experimental import pallas as pl
from jax.experimental.pallas import tpu as pltpu
from jax.experimental.pallas import tpu_sc as plsc

N = 10000
E = 320000
D = 128
R = 4
B = 1024
H = 64

NC = 2
NS = 16
NW = NC * NS

CH = 80
SH = 63
S = 2 * SH
EPW = S * CH
E_PAD = EPW * NW
N_PAD = 10240
ROWS_PW = N_PAD // NS
NBUF = 3
ZR = 64
K0 = 2


def _mesh():
    return plsc.VectorSubcoreMesh(core_axis_name="c", subcore_axis_name="s")


@functools.partial(
    pl.kernel, mesh=_mesh(),
    out_type=jax.ShapeDtypeStruct((NC * N_PAD, D), jnp.float32),
    scratch_types=(
        [pltpu.VMEM((SH, CH), jnp.int32),
         pltpu.VMEM((SH, CH), jnp.int32),
         pltpu.VMEM((NBUF * CH, D), jnp.float32),
         pltpu.VMEM_SHARED((N_PAD, D), jnp.float32)]
        + [pltpu.SemaphoreType.DMA] * (2 * NBUF)
    ),
)
def _agg(table, gidx3, dst3, acc_out, gidx_v, dst_v, rows_v, acc_sh, *sems):
    semg = sems[:NBUF]
    sems_ = sems[NBUF:]
    cid = lax.axis_index("c")
    sid = lax.axis_index("s")

    def zrow(j, c):
        for k in range(D // 16):
            rows_v[j, pl.ds(k * 16, 16)] = jnp.zeros((16,), jnp.float32)
        return c
    lax.fori_loop(0, ZR, zrow, 0)
    base = sid * ROWS_PW
    for t in range(ROWS_PW // ZR):
        pltpu.sync_copy(rows_v.at[pl.ds(0, ZR)],
                        acc_sh.at[pl.ds(base + t * ZR, ZR)])
    plsc.subcore_barrier()

    def buf(b):
        return rows_v.at[pl.ds(b * CH, CH)]

    def fire_g(b, j):
        pltpu.async_copy(table.at[gidx_v.at[j]], buf(b), semg[b])

    def wait_g(b, j):
        pltpu.make_async_copy(table.at[gidx_v.at[j]], buf(b), semg[b]).wait()

    def fire_s(b, j):
        pltpu.async_copy(buf(b), acc_sh.at[dst_v.at[j]], sems_[b], add=True)

    def wait_s(b, j):
        pltpu.make_async_copy(buf(b), acc_sh.at[dst_v.at[j]],
                              sems_[b]).wait()

    def run_slice(hidx):
        pltpu.sync_copy(gidx3.at[hidx], gidx_v)
        pltpu.sync_copy(dst3.at[hidx], dst_v)
        for b in range(NBUF):
            fire_g(b, b)

        def group(t, c):
            for b in range(NBUF):
                j = t * NBUF + b
                wait_g(b, j)
                fire_s(b, j)
            for b in range(NBUF):
                jn = (t + 1) * NBUF + b
                wait_s(b, jn)
                fire_g(b, jn)
            return c
        lax.fori_loop(0, SH // NBUF - 1, group, 0)

        for b in range(NBUF):
            j = SH - NBUF + b
            wait_g(b, j)
            fire_s(b, j)
        for b in range(NBUF):
            wait_s(b, 0)

    @pl.when(cid == 0)
    def _core0():
        for q in range(K0):
            run_slice(sid * K0 + q)

    if K0 < 4:
        @pl.when(cid == 1)
        def _core1():
            for q in range(4 - K0):
                run_slice(NS * K0 + sid * (4 - K0) + q)

    plsc.subcore_barrier()

    pltpu.sync_copy(acc_sh.at[pl.ds(base, ROWS_PW)],
                    acc_out.at[pl.ds(cid * N_PAD + base, ROWS_PW)])


@functools.partial(
    pl.kernel, mesh=_mesh(),
    out_type=jax.ShapeDtypeStruct((NC * N_PAD,), jnp.float32),
    scratch_types=[
        pltpu.VMEM((S, CH), jnp.int32),
        pltpu.VMEM((CH,), jnp.float32),
        pltpu.VMEM((ZR,), jnp.float32),
        pltpu.VMEM_SHARED((N_PAD,), jnp.float32),
        pltpu.SemaphoreType.DMA,
    ],
)
def _deg(dst3, deg_out, dst_v, ones_v, zeros_v, deg_sh, semd):
    cid = lax.axis_index("c")
    sid = lax.axis_index("s")
    wid = sid * NC + cid
    for h in range(2):
        pltpu.sync_copy(dst3.at[2 * wid + h], dst_v.at[pl.ds(h * SH, SH)])
    for k in range(CH // 16):
        ones_v[pl.ds(k * 16, 16)] = jnp.ones((16,), jnp.float32)
    for k in range(ZR // 16):
        zeros_v[pl.ds(k * 16, 16)] = jnp.zeros((16,), jnp.float32)
    base = sid * ROWS_PW
    for t in range(ROWS_PW // ZR):
        pltpu.sync_copy(zeros_v, deg_sh.at[pl.ds(base + t * ZR, ZR)])
    plsc.subcore_barrier()

    def step(j, c):
        pltpu.async_copy(ones_v, deg_sh.at[dst_v.at[j]], semd, add=True)
        return c
    lax.fori_loop(0, S, step, 0)

    def drain(j, c):
        pltpu.make_async_copy(ones_v, deg_sh.at[dst_v.at[0]], semd).wait()
        return c
    lax.fori_loop(0, S, drain, 0)

    plsc.subcore_barrier()
    pltpu.sync_copy(deg_sh.at[pl.ds(base, ROWS_PW)],
                    deg_out.at[pl.ds(cid * N_PAD + base, ROWS_PW)])


_BPW = (2 * B) // NW


@functools.partial(
    pl.kernel, mesh=_mesh(),
    out_type=(jax.ShapeDtypeStruct((3 * 2 * B, D), jnp.float32),
              jax.ShapeDtypeStruct((2 * B,), jnp.float32)),
    scratch_types=[
        pltpu.VMEM((_BPW,), jnp.int32),
        pltpu.VMEM((_BPW,), jnp.int32),
        pltpu.VMEM((3 * _BPW, D), jnp.float32),
        pltpu.VMEM((_BPW,), jnp.float32),
        pltpu.SemaphoreType.DMA,
        pltpu.SemaphoreType.DMA,
    ],
)
def _pair_gather(acc, hs2, rdeg, idx, out3, rdeg_out, idx_v, idx2_v, rows_v,
                 rdeg_v, sem, semd):
    wid = lax.axis_index("s") * NC + lax.axis_index("c")
    base = wid * _BPW
    pltpu.sync_copy(idx.at[pl.ds(base, _BPW)], idx_v)
    for k in range(_BPW // 16):
        sl = pl.ds(k * 16, 16)
        idx2_v[sl] = idx_v[sl] + N_PAD
    pltpu.async_copy(acc.at[idx_v], rows_v.at[pl.ds(0, _BPW)], sem)
    pltpu.async_copy(acc.at[idx2_v], rows_v.at[pl.ds(_BPW, _BPW)], sem)
    pltpu.async_copy(hs2.at[idx_v], rows_v.at[pl.ds(2 * _BPW, _BPW)], sem)
    pltpu.async_copy(rdeg.at[idx_v], rdeg_v, semd).wait()
    pltpu.make_async_copy(acc.at[idx_v], rows_v.at[pl.ds(0, _BPW)], sem).wait()
    pltpu.make_async_copy(acc.at[idx_v], rows_v.at[pl.ds(0, _BPW)], sem).wait()
    pltpu.make_async_copy(acc.at[idx_v], rows_v.at[pl.ds(0, _BPW)], sem).wait()
    for q in range(3):
        pltpu.sync_copy(rows_v.at[pl.ds(q * _BPW, _BPW)],
                        out3.at[pl.ds(q * 2 * B + base, _BPW)])
    pltpu.sync_copy(rdeg_v, rdeg_out.at[pl.ds(base, _BPW)])


def _transform_body(x_ref, wr_ref, ws_ref, hrel_ref, hs_ref):
    xb = x_ref[...]
    for r in range(R):
        hrel_ref[r] = jnp.dot(xb, wr_ref[r], preferred_element_type=jnp.float32)
    hs_ref[...] = jnp.dot(xb, ws_ref[...], preferred_element_type=jnp.float32)


def _tc_transform(x, W_rel, W_self):
    bn = 2000
    return pl.pallas_call(
        _transform_body,
        grid=(N // bn,),
        in_specs=[
            pl.BlockSpec((bn, D), lambda i: (i, 0)),
            pl.BlockSpec((R, D, D), lambda i: (0, 0, 0)),
            pl.BlockSpec((D, D), lambda i: (0, 0)),
        ],
        out_specs=[
            pl.BlockSpec((R, bn, D), lambda i: (0, i, 0)),
            pl.BlockSpec((bn, D), lambda i: (i, 0)),
        ],
        out_shape=[
            jax.ShapeDtypeStruct((R, N, D), jnp.float32),
            jax.ShapeDtypeStruct((N, D), jnp.float32),
        ],
    )(x, W_rel, W_self)


def _combine1_body(acc_ref, deg_ref, hs1_ref, wr2_ref, ws2_ref,
                   hrel2_ref, hs2_ref, rdeg_ref):
    rdeg = 1.0 / jnp.maximum(deg_ref[0] + deg_ref[1], 1.0)
    rdeg_ref[...] = rdeg[None, :]
    agg = acc_ref[0] + acc_ref[1]
    h = jnp.maximum(agg * rdeg[:, None] + hs1_ref[...], 0.0)
    for r in range(R):
        hrel2_ref[r] = jnp.dot(h, wr2_ref[r], preferred_element_type=jnp.float32)
    hs2_ref[...] = jnp.dot(h, ws2_ref[...], preferred_element_type=jnp.float32)


def _tc_combine1(acc, deg, hs1, W_rel2, W_self2):
    bn = 2048
    g = N_PAD // bn
    return pl.pallas_call(
        _combine1_body,
        grid=(g,),
        in_specs=[
            pl.BlockSpec((NC, bn, D), lambda i: (0, i, 0)),
            pl.BlockSpec((NC, bn), lambda i: (0, i)),
            pl.BlockSpec((bn, D), lambda i: (i, 0)),
            pl.BlockSpec((R, D, D), lambda i: (0, 0, 0)),
            pl.BlockSpec((D, D), lambda i: (0, 0)),
        ],
        out_specs=[
            pl.BlockSpec((R, bn, D), lambda i: (0, i, 0)),
            pl.BlockSpec((bn, D), lambda i: (i, 0)),
            pl.BlockSpec((1, bn), lambda i: (0, i)),
        ],
        out_shape=[
            jax.ShapeDtypeStruct((R, N, D), jnp.float32),
            jax.ShapeDtypeStruct((N, D), jnp.float32),
            jax.ShapeDtypeStruct((1, N_PAD), jnp.float32),
        ],
    )(acc, deg, hs1, W_rel2, W_self2)


def _head_body(g3_ref, rdeg_ref, wa_ref, va_ref, wm1_ref, bm1_ref, wm2_ref,
               bm2_ref, out_ref):
    h2 = (g3_ref[0] + g3_ref[1]) * rdeg_ref[...] + g3_ref[2]
    e = h2.reshape(B, 2 * D)
    s = jnp.tanh(jnp.dot(e, wa_ref[...], preferred_element_type=jnp.float32))
    score = jnp.dot(s, va_ref[...], preferred_element_type=jnp.float32)
    gate = jax.nn.sigmoid(score)
    w = gate * e
    o = jnp.dot(w, wm1_ref[...], preferred_element_type=jnp.float32)
    o = jnp.maximum(o + bm1_ref[...], 0.0)
    o = jnp.dot(o, wm2_ref[...], preferred_element_type=jnp.float32)
    o = o + bm2_ref[...]
    out_ref[...] = jnp.mean(o, axis=0, keepdims=True)


def _tc_head(g3, rdeg, Wa, va, Wm1, bm1, Wm2, bm2):
    return pl.pallas_call(
        _head_body,
        out_shape=jax.ShapeDtypeStruct((1, 1), jnp.float32),
    )(g3, rdeg, Wa, va, Wm1, bm1, Wm2, bm2)


def kernel(x, edge_index, edge_type, user_item_pairs,
           W_rel1, W_self1, W_rel2, W_self2, Wa, va, Wm1, bm1, Wm2, bm2):
    src = edge_index[0].astype(jnp.int32)
    dst = edge_index[1].astype(jnp.int32)
    et = edge_type.astype(jnp.int32)
    gidx = et * N + src
    pad = E_PAD - E
    gpad = jnp.arange(pad, dtype=jnp.int32) % N
    jpad = N + jnp.arange(pad, dtype=jnp.int32) % (N_PAD - N)
    gidx3 = jnp.concatenate([gidx, gpad]).reshape(2 * NW, SH, CH)
    dst3 = jnp.concatenate([dst, jpad]).reshape(2 * NW, SH, CH)

    hrel1, hs1 = _tc_transform(x, W_rel1, W_self1)
    deg = _deg(dst3).reshape(NC, N_PAD)
    acc1 = _agg(hrel1.reshape(R * N, D), gidx3, dst3)
    acc1 = acc1.reshape(NC, N_PAD, D)
    hrel2, hs2, rdeg = _tc_combine1(acc1, deg, hs1, W_rel2, W_self2)
    acc2 = _agg(hrel2.reshape(R * N, D), gidx3, dst3)

    pidx = user_item_pairs.astype(jnp.int32).reshape(2 * B)
    g3, prdeg = _pair_gather(acc2, hs2, rdeg.reshape(N_PAD), pidx)
    out = _tc_head(g3.reshape(3, 2 * B, D), prdeg.reshape(2 * B, 1),
                   Wa, va.reshape(H, 1), Wm1, bm1.reshape(1, H),
                   Wm2, bm2.reshape(1, 1))
    return out.reshape(1)


# speedup vs baseline: 4.2249x; 1.0394x over previous
import functools

import jax
import jax.numpy as jnp
from jax import lax
from jax.experimental import pallas as pl
from jax.experimental.pallas import tpu as pltpu
from jax.experimental.pallas import tpu_sc as plsc

N = 10000
E = 320000
D = 128
R = 4
B = 1024
H = 64

NC = 2
NS = 16
NW = NC * NS

CH = 64
SH = 40
SL_PW = 4
S = SL_PW * SH
EPW = S * CH
E_PAD = EPW * NW
N_PAD = 10240
ROWS_PW = N_PAD // NS
NBUF = 4
ZR = 64


def _mesh():
    return plsc.VectorSubcoreMesh(core_axis_name="c", subcore_axis_name="s")


@functools.partial(
    pl.kernel, mesh=_mesh(),
    out_type=jax.ShapeDtypeStruct((NC * N_PAD, D), jnp.float32),
    scratch_types=(
        [pltpu.VMEM((SH, CH), jnp.int32),
         pltpu.VMEM((SH, CH), jnp.int32),
         pltpu.VMEM((NBUF * CH, D), jnp.float32),
         pltpu.VMEM_SHARED((N_PAD, D), jnp.float32)]
        + [pltpu.SemaphoreType.DMA] * (2 * NBUF)
    ),
)
def _agg(table, gidx3, dst3, acc_out, gidx_v, dst_v, rows_v, acc_sh, *sems):
    semg = sems[:NBUF]
    sems_ = sems[NBUF:]
    cid = lax.axis_index("c")
    sid = lax.axis_index("s")

    def zrow(j, c):
        for k in range(D // 16):
            rows_v[j, pl.ds(k * 16, 16)] = jnp.zeros((16,), jnp.float32)
        return c
    lax.fori_loop(0, ZR, zrow, 0)
    base = sid * ROWS_PW
    for t in range(ROWS_PW // ZR):
        pltpu.sync_copy(rows_v.at[pl.ds(0, ZR)],
                        acc_sh.at[pl.ds(base + t * ZR, ZR)])
    plsc.subcore_barrier()

    def buf(b):
        return rows_v.at[pl.ds(b * CH, CH)]

    def fire_g(b, j):
        pltpu.async_copy(table.at[gidx_v.at[j]], buf(b), semg[b])

    def wait_g(b, j):
        pltpu.make_async_copy(table.at[gidx_v.at[j]], buf(b), semg[b]).wait()

    def fire_s(b, j):
        pltpu.async_copy(buf(b), acc_sh.at[dst_v.at[j]], sems_[b], add=True)

    def wait_s(b, j):
        pltpu.make_async_copy(buf(b), acc_sh.at[dst_v.at[j]],
                              sems_[b]).wait()

    def run_slice(hidx):
        pltpu.sync_copy(gidx3.at[hidx], gidx_v)
        pltpu.sync_copy(dst3.at[hidx], dst_v)
        for b in range(NBUF):
            fire_g(b, b)

        def group(t, c):
            for b in range(NBUF):
                j = t * NBUF + b
                wait_g(b, j)
                fire_s(b, j)
            for b in range(NBUF):
                jn = (t + 1) * NBUF + b
                wait_s(b, jn)
                fire_g(b, jn)
            return c
        lax.fori_loop(0, SH // NBUF - 1, group, 0)

        for b in range(NBUF):
            j = SH - NBUF + b
            wait_g(b, j)
            fire_s(b, j)
        for b in range(NBUF):
            wait_s(b, 0)

    wid2 = cid * NS + sid
    for q in range(SL_PW):
        run_slice(wid2 * SL_PW + q)

    plsc.subcore_barrier()

    pltpu.sync_copy(acc_sh.at[pl.ds(base, ROWS_PW)],
                    acc_out.at[pl.ds(cid * N_PAD + base, ROWS_PW)])


@functools.partial(
    pl.kernel, mesh=_mesh(),
    out_type=jax.ShapeDtypeStruct((NC * N_PAD,), jnp.float32),
    scratch_types=[
        pltpu.VMEM((S, CH), jnp.int32),
        pltpu.VMEM((CH,), jnp.float32),
        pltpu.VMEM((ZR,), jnp.float32),
        pltpu.VMEM_SHARED((N_PAD,), jnp.float32),
        pltpu.SemaphoreType.DMA,
    ],
)
def _deg(dst3, deg_out, dst_v, ones_v, zeros_v, deg_sh, semd):
    cid = lax.axis_index("c")
    sid = lax.axis_index("s")
    wid = sid * NC + cid
    for h in range(SL_PW):
        pltpu.sync_copy(dst3.at[SL_PW * wid + h], dst_v.at[pl.ds(h * SH, SH)])
    for k in range(CH // 16):
        ones_v[pl.ds(k * 16, 16)] = jnp.ones((16,), jnp.float32)
    for k in range(ZR // 16):
        zeros_v[pl.ds(k * 16, 16)] = jnp.zeros((16,), jnp.float32)
    base = sid * ROWS_PW
    for t in range(ROWS_PW // ZR):
        pltpu.sync_copy(zeros_v, deg_sh.at[pl.ds(base + t * ZR, ZR)])
    plsc.subcore_barrier()

    def step(j, c):
        pltpu.async_copy(ones_v, deg_sh.at[dst_v.at[j]], semd, add=True)
        return c
    lax.fori_loop(0, S, step, 0)

    def drain(j, c):
        pltpu.make_async_copy(ones_v, deg_sh.at[dst_v.at[0]], semd).wait()
        return c
    lax.fori_loop(0, S, drain, 0)

    plsc.subcore_barrier()
    pltpu.sync_copy(deg_sh.at[pl.ds(base, ROWS_PW)],
                    deg_out.at[pl.ds(cid * N_PAD + base, ROWS_PW)])


_BPW = (2 * B) // NW


@functools.partial(
    pl.kernel, mesh=_mesh(),
    out_type=(jax.ShapeDtypeStruct((3 * 2 * B, D), jnp.float32),
              jax.ShapeDtypeStruct((2 * B,), jnp.float32)),
    scratch_types=[
        pltpu.VMEM((_BPW,), jnp.int32),
        pltpu.VMEM((_BPW,), jnp.int32),
        pltpu.VMEM((3 * _BPW, D), jnp.float32),
        pltpu.VMEM((_BPW,), jnp.float32),
        pltpu.SemaphoreType.DMA,
        pltpu.SemaphoreType.DMA,
    ],
)
def _pair_gather(acc, hs2, rdeg, idx, out3, rdeg_out, idx_v, idx2_v, rows_v,
                 rdeg_v, sem, semd):
    wid = lax.axis_index("s") * NC + lax.axis_index("c")
    base = wid * _BPW
    pltpu.sync_copy(idx.at[pl.ds(base, _BPW)], idx_v)
    for k in range(_BPW // 16):
        sl = pl.ds(k * 16, 16)
        idx2_v[sl] = idx_v[sl] + N_PAD
    pltpu.async_copy(acc.at[idx_v], rows_v.at[pl.ds(0, _BPW)], sem)
    pltpu.async_copy(acc.at[idx2_v], rows_v.at[pl.ds(_BPW, _BPW)], sem)
    pltpu.async_copy(hs2.at[idx_v], rows_v.at[pl.ds(2 * _BPW, _BPW)], sem)
    pltpu.async_copy(rdeg.at[idx_v], rdeg_v, semd).wait()
    pltpu.make_async_copy(acc.at[idx_v], rows_v.at[pl.ds(0, _BPW)], sem).wait()
    pltpu.make_async_copy(acc.at[idx_v], rows_v.at[pl.ds(0, _BPW)], sem).wait()
    pltpu.make_async_copy(acc.at[idx_v], rows_v.at[pl.ds(0, _BPW)], sem).wait()
    for q in range(3):
        pltpu.sync_copy(rows_v.at[pl.ds(q * _BPW, _BPW)],
                        out3.at[pl.ds(q * 2 * B + base, _BPW)])
    pltpu.sync_copy(rdeg_v, rdeg_out.at[pl.ds(base, _BPW)])


def _transform_body(x_ref, wr_ref, ws_ref, hrel_ref, hs_ref):
    xb = x_ref[...]
    for r in range(R):
        hrel_ref[r] = jnp.dot(xb, wr_ref[r], preferred_element_type=jnp.float32)
    hs_ref[...] = jnp.dot(xb, ws_ref[...], preferred_element_type=jnp.float32)


def _tc_transform(x, W_rel, W_self):
    bn = 2000
    return pl.pallas_call(
        _transform_body,
        grid=(N // bn,),
        in_specs=[
            pl.BlockSpec((bn, D), lambda i: (i, 0)),
            pl.BlockSpec((R, D, D), lambda i: (0, 0, 0)),
            pl.BlockSpec((D, D), lambda i: (0, 0)),
        ],
        out_specs=[
            pl.BlockSpec((R, bn, D), lambda i: (0, i, 0)),
            pl.BlockSpec((bn, D), lambda i: (i, 0)),
        ],
        out_shape=[
            jax.ShapeDtypeStruct((R, N, D), jnp.float32),
            jax.ShapeDtypeStruct((N, D), jnp.float32),
        ],
    )(x, W_rel, W_self)


def _combine1_body(acc_ref, deg_ref, hs1_ref, wr2_ref, ws2_ref,
                   hrel2_ref, hs2_ref, rdeg_ref):
    rdeg = 1.0 / jnp.maximum(deg_ref[0] + deg_ref[1], 1.0)
    rdeg_ref[...] = rdeg[None, :]
    agg = acc_ref[0] + acc_ref[1]
    h = jnp.maximum(agg * rdeg[:, None] + hs1_ref[...], 0.0)
    for r in range(R):
        hrel2_ref[r] = jnp.dot(h, wr2_ref[r], preferred_element_type=jnp.float32)
    hs2_ref[...] = jnp.dot(h, ws2_ref[...], preferred_element_type=jnp.float32)


def _tc_combine1(acc, deg, hs1, W_rel2, W_self2):
    bn = 2048
    g = N_PAD // bn
    return pl.pallas_call(
        _combine1_body,
        grid=(g,),
        in_specs=[
            pl.BlockSpec((NC, bn, D), lambda i: (0, i, 0)),
            pl.BlockSpec((NC, bn), lambda i: (0, i)),
            pl.BlockSpec((bn, D), lambda i: (i, 0)),
            pl.BlockSpec((R, D, D), lambda i: (0, 0, 0)),
            pl.BlockSpec((D, D), lambda i: (0, 0)),
        ],
        out_specs=[
            pl.BlockSpec((R, bn, D), lambda i: (0, i, 0)),
            pl.BlockSpec((bn, D), lambda i: (i, 0)),
            pl.BlockSpec((1, bn), lambda i: (0, i)),
        ],
        out_shape=[
            jax.ShapeDtypeStruct((R, N, D), jnp.float32),
            jax.ShapeDtypeStruct((N, D), jnp.float32),
            jax.ShapeDtypeStruct((1, N_PAD), jnp.float32),
        ],
    )(acc, deg, hs1, W_rel2, W_self2)


def _head_body(g3_ref, rdeg_ref, wa_ref, va_ref, wm1_ref, bm1_ref, wm2_ref,
               bm2_ref, out_ref):
    h2 = (g3_ref[0] + g3_ref[1]) * rdeg_ref[...] + g3_ref[2]
    e = h2.reshape(B, 2 * D)
    s = jnp.tanh(jnp.dot(e, wa_ref[...], preferred_element_type=jnp.float32))
    score = jnp.dot(s, va_ref[...], preferred_element_type=jnp.float32)
    gate = jax.nn.sigmoid(score)
    w = gate * e
    o = jnp.dot(w, wm1_ref[...], preferred_element_type=jnp.float32)
    o = jnp.maximum(o + bm1_ref[...], 0.0)
    o = jnp.dot(o, wm2_ref[...], preferred_element_type=jnp.float32)
    o = o + bm2_ref[...]
    out_ref[...] = jnp.mean(o, axis=0, keepdims=True)


def _tc_head(g3, rdeg, Wa, va, Wm1, bm1, Wm2, bm2):
    return pl.pallas_call(
        _head_body,
        out_shape=jax.ShapeDtypeStruct((1, 1), jnp.float32),
    )(g3, rdeg, Wa, va, Wm1, bm1, Wm2, bm2)


def kernel(x, edge_index, edge_type, user_item_pairs,
           W_rel1, W_self1, W_rel2, W_self2, Wa, va, Wm1, bm1, Wm2, bm2):
    src = edge_index[0].astype(jnp.int32)
    dst = edge_index[1].astype(jnp.int32)
    et = edge_type.astype(jnp.int32)
    gidx = et * N + src
    pad = E_PAD - E
    gpad = jnp.arange(pad, dtype=jnp.int32) % N
    jpad = N + jnp.arange(pad, dtype=jnp.int32) % (N_PAD - N)
    gidx3 = jnp.concatenate([gidx, gpad]).reshape(SL_PW * NW, SH, CH)
    dst3 = jnp.concatenate([dst, jpad]).reshape(SL_PW * NW, SH, CH)

    hrel1, hs1 = _tc_transform(x, W_rel1, W_self1)
    deg = _deg(dst3).reshape(NC, N_PAD)
    acc1 = _agg(hrel1.reshape(R * N, D), gidx3, dst3)
    acc1 = acc1.reshape(NC, N_PAD, D)
    hrel2, hs2, rdeg = _tc_combine1(acc1, deg, hs1, W_rel2, W_self2)
    acc2 = _agg(hrel2.reshape(R * N, D), gidx3, dst3)

    pidx = user_item_pairs.astype(jnp.int32).reshape(2 * B)
    g3, prdeg = _pair_gather(acc2, hs2, rdeg.reshape(N_PAD), pidx)
    out = _tc_head(g3.reshape(3, 2 * B, D), prdeg.reshape(2 * B, 1),
                   Wa, va.reshape(H, 1), Wm1, bm1.reshape(1, H),
                   Wm2, bm2.reshape(1, 1))
    return out.reshape(1)
